# baseline (speedup 1.0000x reference)
import functools

import jax
import jax.numpy as jnp
from jax import lax
from jax.experimental import pallas as pl
from jax.experimental.pallas import tpu as pltpu
from jax.experimental.pallas import tpu_sc as plsc

R = 128
N = 32768
K_SEL = 32
L = 16
NC = 2
NS = 16
NW = NC * NS
ROWS_PW = R // NW
CHUNK = 8192
NCHUNK = N // CHUNK
VPB = 16
NBLK = CHUNK // (L * VPB)

_GDN = lax.GatherDimensionNumbers(
    offset_dims=(), collapsed_slice_dims=(0,), start_index_map=(0,)
)


def _lane():
    return lax.iota(jnp.int32, L)


def _make_idx():
    lane = _lane()
    X = {j: (lane ^ j)[:, None] for j in (1, 2, 4, 8)}
    sortpre = []
    sp = 1
    for p in (2, 4, 8, 16):
        j = p // 2
        sj = sp - 1
        while j >= 1:
            sortpre.append((j, ((lane >> sj) ^ (lane >> sp)) & 1))
            j //= 2
            sj -= 1
        sp += 1
    X["sortpre"] = tuple(sortpre)
    X["mergepre"] = tuple((j, lane & j) for j in (8, 4, 2, 1))
    return X


def _shuffle(X, v, j):
    return lax.gather(
        v, X[j], _GDN, (1,), mode=lax.GatherScatterMode.PROMISE_IN_BOUNDS
    )


def _ce(X, v, j, take_min):
    pv = _shuffle(X, v, j)
    return jnp.where(take_min, jnp.minimum(v, pv), jnp.maximum(v, pv))


def _sort16(X, v):
    for j, pre in X["sortpre"]:
        v = _ce(X, v, j, pre == 0)
    return v


def _bitonic_merge16(X, v):
    for j, pre in X["mergepre"]:
        v = _ce(X, v, j, pre == 0)
    return v


def _lane_max(X, v):
    for j in (8, 4, 2, 1):
        v = jnp.maximum(v, _shuffle(X, v, j))
    return v


def _lane_or(X, v):
    for j in (8, 4, 2, 1):
        v = v | _shuffle(X, v, j)
    return v


def _lane_sum(X, v):
    for j in (8, 4, 2, 1):
        v = v + _shuffle(X, v, j)
    return v


def _merge_topk(X, a0, a1, v):
    rv = lax.rev(_sort16(X, v), (0,))
    mlo = jnp.maximum(a0, rv)
    n0 = jnp.minimum(mlo, a1)
    n1 = jnp.maximum(mlo, a1)
    return _bitonic_merge16(X, n0), _bitonic_merge16(X, n1)


_sc_mesh = plsc.VectorSubcoreMesh(core_axis_name="c", subcore_axis_name="s")


@functools.partial(
    pl.kernel,
    out_type=jax.ShapeDtypeStruct((NW * L,), jnp.float32),
    mesh=_sc_mesh,
    scratch_types=[
        pltpu.VMEM((CHUNK,), jnp.float32),
        pltpu.VMEM((CHUNK,), jnp.float32),
        pltpu.VMEM((2 * L,), jnp.float32),
        pltpu.VMEM((L,), jnp.float32),
        pltpu.SemaphoreType.DMA,
        pltpu.SemaphoreType.DMA,
    ],
)
def _topk_mean_sc(x_hbm, out_hbm, buf0, buf1, topv, means_v, sem0, sem1):
    cid = lax.axis_index("c")
    sid = lax.axis_index("s")
    wid = sid * NC + cid
    X = _make_idx()

    means_v[...] = jnp.zeros((L,), jnp.float32)
    bufs = (buf0, buf1)
    sems = (sem0, sem1)

    def _run_chunk(buf, thr0):
        def blk_fn(b, thr):
            base = b * (VPB * L)
            vs = [buf[pl.ds(base + j * L, L)] for j in range(VPB)]
            t = vs
            while len(t) > 1:
                t = [jnp.maximum(t[i], t[i + 1]) for i in range(0, len(t), 2)]
            hit = _lane_max(X, t[0])[0] > thr

            def _process(thr_in):
                thr_splat = jnp.full((L,), thr_in, jnp.float32)
                bmv = jnp.zeros((L,), jnp.int32)
                for j in range(VPB):
                    v = buf[pl.ds(base + j * L, L)]
                    bmv = bmv | jnp.where(
                        v > thr_splat, jnp.int32(1 << j), jnp.int32(0)
                    )
                bm = _lane_or(X, bmv)[0]

                thr_cur = thr_in
                for j in range(VPB):
                    def _merge_j(t, j=j):
                        del t
                        v = buf[pl.ds(base + j * L, L)]
                        a0 = topv[pl.ds(0, L)]
                        a1 = topv[pl.ds(L, L)]
                        n0, n1 = _merge_topk(X, a0, a1, v)
                        topv[pl.ds(0, L)] = n0
                        topv[pl.ds(L, L)] = n1
                        return n0[0]

                    flagged = ((bm >> j) & 1) != 0
                    thr_cur = lax.cond(flagged, _merge_j, lambda t: t, thr_cur)
                return thr_cur

            return lax.cond(hit, _process, lambda t: t, thr)

        return lax.fori_loop(0, NBLK, blk_fn, thr0)

    def row_fn(r, carry):
        rowbase = (wid * ROWS_PW + r) * N
        neg = jnp.full((L,), -jnp.inf, jnp.float32)
        topv[pl.ds(0, L)] = neg
        topv[pl.ds(L, L)] = neg

        copies = [None] * NCHUNK
        copies[0] = pltpu.async_copy(
            x_hbm.at[pl.ds(rowbase, CHUNK)], bufs[0], sems[0]
        )
        thr = jnp.float32(-jnp.inf)
        for c in range(NCHUNK):
            copies[c].wait()
            if c + 1 < NCHUNK:
                copies[c + 1] = pltpu.async_copy(
                    x_hbm.at[pl.ds(rowbase + (c + 1) * CHUNK, CHUNK)],
                    bufs[(c + 1) % 2],
                    sems[(c + 1) % 2],
                )
            thr = _run_chunk(bufs[c % 2], thr)

        a0 = topv[pl.ds(0, L)]
        a1 = topv[pl.ds(L, L)]
        mean = _lane_sum(X, a0 + a1) * jnp.float32(1.0 / K_SEL)
        means_v[...] = jnp.where(_lane() == r, mean, means_v[...])
        return carry

    lax.fori_loop(0, ROWS_PW, row_fn, 0)
    pltpu.sync_copy(means_v, out_hbm.at[pl.ds(wid * L, L)])


def kernel(x):
    out = _topk_mean_sc(x.reshape(R * N))
    return out.reshape(NW, L)[:, :ROWS_PW].reshape(R)


# speedup vs baseline: 1.0534x; 1.0534x over previous
import functools

import jax
import jax.numpy as jnp
from jax import lax
from jax.experimental import pallas as pl
from jax.experimental.pallas import tpu as pltpu
from jax.experimental.pallas import tpu_sc as plsc

R = 128
N = 32768
K_SEL = 32
L = 16
NC = 2
NS = 16
NW = NC * NS
ROWS_PW = R // NW
CHUNK = 8192
NCHUNK = N // CHUNK
VPB = 8
NBLK = CHUNK // (L * VPB)

_GDN = lax.GatherDimensionNumbers(
    offset_dims=(), collapsed_slice_dims=(0,), start_index_map=(0,)
)


def _lane():
    return lax.iota(jnp.int32, L)


def _make_idx():
    lane = _lane()
    X = {j: (lane ^ j)[:, None] for j in (1, 2, 4, 8)}
    sortpre = []
    sp = 1
    for p in (2, 4, 8, 16):
        j = p // 2
        sj = sp - 1
        while j >= 1:
            sortpre.append((j, ((lane >> sj) ^ (lane >> sp)) & 1))
            j //= 2
            sj -= 1
        sp += 1
    X["sortpre"] = tuple(sortpre)
    X["mergepre"] = tuple((j, lane & j) for j in (8, 4, 2, 1))
    return X


def _shuffle(X, v, j):
    return lax.gather(
        v, X[j], _GDN, (1,), mode=lax.GatherScatterMode.PROMISE_IN_BOUNDS
    )


def _ce(X, v, j, take_min):
    pv = _shuffle(X, v, j)
    return jnp.where(take_min, jnp.minimum(v, pv), jnp.maximum(v, pv))


def _sort16(X, v):
    for j, pre in X["sortpre"]:
        v = _ce(X, v, j, pre == 0)
    return v


def _bitonic_merge16(X, v):
    for j, pre in X["mergepre"]:
        v = _ce(X, v, j, pre == 0)
    return v


def _lane_max(X, v):
    for j in (8, 4, 2, 1):
        v = jnp.maximum(v, _shuffle(X, v, j))
    return v


def _lane_or(X, v):
    for j in (8, 4, 2, 1):
        v = v | _shuffle(X, v, j)
    return v


def _lane_sum(X, v):
    for j in (8, 4, 2, 1):
        v = v + _shuffle(X, v, j)
    return v


def _merge_topk(X, a0, a1, v):
    rv = lax.rev(_sort16(X, v), (0,))
    mlo = jnp.maximum(a0, rv)
    n0 = jnp.minimum(mlo, a1)
    n1 = jnp.maximum(mlo, a1)
    return _bitonic_merge16(X, n0), _bitonic_merge16(X, n1)


_sc_mesh = plsc.VectorSubcoreMesh(core_axis_name="c", subcore_axis_name="s")


@functools.partial(
    pl.kernel,
    out_type=jax.ShapeDtypeStruct((NW * L,), jnp.float32),
    mesh=_sc_mesh,
    scratch_types=[
        pltpu.VMEM((CHUNK,), jnp.float32),
        pltpu.VMEM((CHUNK,), jnp.float32),
        pltpu.VMEM((2 * L,), jnp.float32),
        pltpu.VMEM((L,), jnp.float32),
        pltpu.SemaphoreType.DMA,
        pltpu.SemaphoreType.DMA,
    ],
)
def _topk_mean_sc(x_hbm, out_hbm, buf0, buf1, topv, means_v, sem0, sem1):
    cid = lax.axis_index("c")
    sid = lax.axis_index("s")
    wid = sid * NC + cid
    X = _make_idx()

    means_v[...] = jnp.zeros((L,), jnp.float32)
    bufs = (buf0, buf1)
    sems = (sem0, sem1)

    def _run_chunk(buf, thr0):
        def blk_fn(b, thr):
            base = b * (VPB * L)
            vs = [buf[pl.ds(base + j * L, L)] for j in range(VPB)]
            t = vs
            while len(t) > 1:
                t = [jnp.maximum(t[i], t[i + 1]) for i in range(0, len(t), 2)]
            hit = _lane_max(X, t[0])[0] > thr

            def _process(thr_in):
                thr_splat = jnp.full((L,), thr_in, jnp.float32)
                bmv = jnp.zeros((L,), jnp.int32)
                for j in range(VPB):
                    v = buf[pl.ds(base + j * L, L)]
                    bmv = bmv | jnp.where(
                        v > thr_splat, jnp.int32(1 << j), jnp.int32(0)
                    )
                bm = _lane_or(X, bmv)[0]

                thr_cur = thr_in
                for j in range(VPB):
                    def _merge_j(t, j=j):
                        del t
                        v = buf[pl.ds(base + j * L, L)]
                        a0 = topv[pl.ds(0, L)]
                        a1 = topv[pl.ds(L, L)]
                        n0, n1 = _merge_topk(X, a0, a1, v)
                        topv[pl.ds(0, L)] = n0
                        topv[pl.ds(L, L)] = n1
                        return n0[0]

                    flagged = ((bm >> j) & 1) != 0
                    thr_cur = lax.cond(flagged, _merge_j, lambda t: t, thr_cur)
                return thr_cur

            return lax.cond(hit, _process, lambda t: t, thr)

        return lax.fori_loop(0, NBLK, blk_fn, thr0)

    def row_fn(r, carry):
        rowbase = (wid * ROWS_PW + r) * N
        neg = jnp.full((L,), -jnp.inf, jnp.float32)
        topv[pl.ds(0, L)] = neg
        topv[pl.ds(L, L)] = neg

        copies = [None] * NCHUNK
        copies[0] = pltpu.async_copy(
            x_hbm.at[pl.ds(rowbase, CHUNK)], bufs[0], sems[0]
        )
        thr = jnp.float32(-jnp.inf)
        for c in range(NCHUNK):
            copies[c].wait()
            if c + 1 < NCHUNK:
                copies[c + 1] = pltpu.async_copy(
                    x_hbm.at[pl.ds(rowbase + (c + 1) * CHUNK, CHUNK)],
                    bufs[(c + 1) % 2],
                    sems[(c + 1) % 2],
                )
            thr = _run_chunk(bufs[c % 2], thr)

        a0 = topv[pl.ds(0, L)]
        a1 = topv[pl.ds(L, L)]
        mean = _lane_sum(X, a0 + a1) * jnp.float32(1.0 / K_SEL)
        means_v[...] = jnp.where(_lane() == r, mean, means_v[...])
        return carry

    lax.fori_loop(0, ROWS_PW, row_fn, 0)
    pltpu.sync_copy(means_v, out_hbm.at[pl.ds(wid * L, L)])


def kernel(x):
    out = _topk_mean_sc(x.reshape(R * N))
    return out.reshape(NW, L)[:, :ROWS_PW].reshape(R)


# speedup vs baseline: 1.2032x; 1.1422x over previous
import functools

import jax
import jax.numpy as jnp
from jax import lax
from jax.experimental import pallas as pl
from jax.experimental.pallas import tpu as pltpu
from jax.experimental.pallas import tpu_sc as plsc

R = 128
N = 32768
K_SEL = 32
L = 16
NC = 2
NS = 16
NW = NC * NS
ROWS_PW = R // NW
CHUNK = 8192
NCHUNK = N // CHUNK
VPB = 16
NBLK = CHUNK // (L * VPB)

_GDN = lax.GatherDimensionNumbers(
    offset_dims=(), collapsed_slice_dims=(0,), start_index_map=(0,)
)


def _lane():
    return lax.iota(jnp.int32, L)


def _make_idx():
    lane = _lane()
    X = {j: (lane ^ j)[:, None] for j in (1, 2, 4, 8)}
    sortpre = []
    sp = 1
    for p in (2, 4, 8, 16):
        j = p // 2
        sj = sp - 1
        while j >= 1:
            sortpre.append((j, ((lane >> sj) ^ (lane >> sp)) & 1))
            j //= 2
            sj -= 1
        sp += 1
    X["sortpre"] = tuple(sortpre)
    X["mergepre"] = tuple((j, lane & j) for j in (8, 4, 2, 1))
    return X


def _shuffle(X, v, j):
    return lax.gather(
        v, X[j], _GDN, (1,), mode=lax.GatherScatterMode.PROMISE_IN_BOUNDS
    )


def _ce(X, v, j, take_min):
    pv = _shuffle(X, v, j)
    return jnp.where(take_min, jnp.minimum(v, pv), jnp.maximum(v, pv))


def _sort16(X, v):
    for j, pre in X["sortpre"]:
        v = _ce(X, v, j, pre == 0)
    return v


def _bitonic_merge16(X, v):
    for j, pre in X["mergepre"]:
        v = _ce(X, v, j, pre == 0)
    return v


def _lane_max(X, v):
    for j in (8, 4, 2, 1):
        v = jnp.maximum(v, _shuffle(X, v, j))
    return v


def _lane_or(X, v):
    for j in (8, 4, 2, 1):
        v = v | _shuffle(X, v, j)
    return v


def _lane_sum(X, v):
    for j in (8, 4, 2, 1):
        v = v + _shuffle(X, v, j)
    return v


def _merge_topk(X, a0, a1, v):
    rv = lax.rev(_sort16(X, v), (0,))
    mlo = jnp.maximum(a0, rv)
    n0 = jnp.minimum(mlo, a1)
    n1 = jnp.maximum(mlo, a1)
    return _bitonic_merge16(X, n0), _bitonic_merge16(X, n1)


_sc_mesh = plsc.VectorSubcoreMesh(core_axis_name="c", subcore_axis_name="s")


@functools.partial(
    pl.kernel,
    out_type=jax.ShapeDtypeStruct((NW * L,), jnp.float32),
    mesh=_sc_mesh,
    scratch_types=[
        pltpu.VMEM((CHUNK,), jnp.float32),
        pltpu.VMEM((CHUNK,), jnp.float32),
        pltpu.VMEM((2 * L,), jnp.float32),
        pltpu.VMEM((L,), jnp.float32),
        pltpu.SemaphoreType.DMA,
        pltpu.SemaphoreType.DMA,
    ],
)
def _topk_mean_sc(x_hbm, out_hbm, buf0, buf1, topv, means_v, sem0, sem1):
    cid = lax.axis_index("c")
    sid = lax.axis_index("s")
    wid = sid * NC + cid
    X = _make_idx()

    means_v[...] = jnp.zeros((L,), jnp.float32)
    bufs = (buf0, buf1)
    sems = (sem0, sem1)

    def _run_chunk(buf, thr0):
        def blk_fn(b, thr):
            base = b * (VPB * L)
            vs = [buf[pl.ds(base + j * L, L)] for j in range(VPB)]
            t = vs
            while len(t) > 1:
                t = [jnp.maximum(t[i], t[i + 1]) for i in range(0, len(t), 2)]
            hit = _lane_max(X, t[0])[0] > thr

            def _process(thr_in):
                thr_splat = jnp.full((L,), thr_in, jnp.float32)
                bmv = jnp.zeros((L,), jnp.int32)
                for j in range(VPB):
                    v = buf[pl.ds(base + j * L, L)]
                    bmv = bmv | jnp.where(
                        v > thr_splat, jnp.int32(1 << j), jnp.int32(0)
                    )
                bm_splat = _lane_or(X, bmv)
                bm = bm_splat[0]
                bits = (bm_splat >> _lane()) & 1
                cnt = _lane_sum(X, bits)[0]

                def w_body(_, st):
                    bm_c, _thr = st
                    lowbit = bm_c & (-bm_c)
                    fbits = lax.bitcast_convert_type(
                        lowbit.astype(jnp.float32), jnp.int32
                    )
                    j = (fbits >> 23) - 127
                    v = buf[pl.ds(base + j * L, L)]
                    a0 = topv[pl.ds(0, L)]
                    a1 = topv[pl.ds(L, L)]
                    n0, n1 = _merge_topk(X, a0, a1, v)
                    topv[pl.ds(0, L)] = n0
                    topv[pl.ds(L, L)] = n1
                    return bm_c & (bm_c - 1), n0[0]

                out = lax.fori_loop(0, cnt, w_body, (bm, thr_in))
                return out[1]

            return lax.cond(hit, _process, lambda t: t, thr)

        return lax.fori_loop(0, NBLK, blk_fn, thr0)

    def row_fn(r, carry):
        rowbase = (wid * ROWS_PW + r) * N
        neg = jnp.full((L,), -jnp.inf, jnp.float32)
        topv[pl.ds(0, L)] = neg
        topv[pl.ds(L, L)] = neg

        copies = [None] * NCHUNK
        copies[0] = pltpu.async_copy(
            x_hbm.at[pl.ds(rowbase, CHUNK)], bufs[0], sems[0]
        )
        thr = jnp.float32(-jnp.inf)
        for c in range(NCHUNK):
            copies[c].wait()
            if c + 1 < NCHUNK:
                copies[c + 1] = pltpu.async_copy(
                    x_hbm.at[pl.ds(rowbase + (c + 1) * CHUNK, CHUNK)],
                    bufs[(c + 1) % 2],
                    sems[(c + 1) % 2],
                )
            thr = _run_chunk(bufs[c % 2], thr)

        a0 = topv[pl.ds(0, L)]
        a1 = topv[pl.ds(L, L)]
        mean = _lane_sum(X, a0 + a1) * jnp.float32(1.0 / K_SEL)
        means_v[...] = jnp.where(_lane() == r, mean, means_v[...])
        return carry

    lax.fori_loop(0, ROWS_PW, row_fn, 0)
    pltpu.sync_copy(means_v, out_hbm.at[pl.ds(wid * L, L)])


def kernel(x):
    out = _topk_mean_sc(x.reshape(R * N))
    return out.reshape(NW, L)[:, :ROWS_PW].reshape(R)


# speedup vs baseline: 1.3771x; 1.1445x over previous
import functools

import jax
import jax.numpy as jnp
from jax import lax
from jax.experimental import pallas as pl
from jax.experimental.pallas import tpu as pltpu
from jax.experimental.pallas import tpu_sc as plsc

R = 128
N = 32768
K_SEL = 32
L = 16
NC = 2
NS = 16
NW = NC * NS
ROWS_PW = R // NW
CHUNK = 8192
NCHUNK = N // CHUNK
VPB = 32
NBLK = CHUNK // (L * VPB)

_GDN = lax.GatherDimensionNumbers(
    offset_dims=(), collapsed_slice_dims=(0,), start_index_map=(0,)
)


def _lane():
    return lax.iota(jnp.int32, L)


def _make_idx():
    lane = _lane()
    X = {j: (lane ^ j)[:, None] for j in (1, 2, 4, 8)}
    sortpre = []
    sp = 1
    for p in (2, 4, 8, 16):
        j = p // 2
        sj = sp - 1
        while j >= 1:
            sortpre.append((j, ((lane >> sj) ^ (lane >> sp)) & 1))
            j //= 2
            sj -= 1
        sp += 1
    X["sortpre"] = tuple(sortpre)
    X["mergepre"] = tuple((j, lane & j) for j in (8, 4, 2, 1))
    return X


def _shuffle(X, v, j):
    return lax.gather(
        v, X[j], _GDN, (1,), mode=lax.GatherScatterMode.PROMISE_IN_BOUNDS
    )


def _ce(X, v, j, take_min):
    pv = _shuffle(X, v, j)
    return jnp.where(take_min, jnp.minimum(v, pv), jnp.maximum(v, pv))


def _sort16(X, v):
    for j, pre in X["sortpre"]:
        v = _ce(X, v, j, pre == 0)
    return v


def _bitonic_merge16(X, v):
    for j, pre in X["mergepre"]:
        v = _ce(X, v, j, pre == 0)
    return v


def _lane_max(X, v):
    for j in (8, 4, 2, 1):
        v = jnp.maximum(v, _shuffle(X, v, j))
    return v


def _lane_or(X, v):
    for j in (8, 4, 2, 1):
        v = v | _shuffle(X, v, j)
    return v


def _lane_sum(X, v):
    for j in (8, 4, 2, 1):
        v = v + _shuffle(X, v, j)
    return v


def _merge_topk(X, a0, a1, v):
    rv = lax.rev(_sort16(X, v), (0,))
    mlo = jnp.maximum(a0, rv)
    n0 = jnp.minimum(mlo, a1)
    n1 = jnp.maximum(mlo, a1)
    return _bitonic_merge16(X, n0), _bitonic_merge16(X, n1)


_sc_mesh = plsc.VectorSubcoreMesh(core_axis_name="c", subcore_axis_name="s")


@functools.partial(
    pl.kernel,
    out_type=jax.ShapeDtypeStruct((NW * L,), jnp.float32),
    mesh=_sc_mesh,
    scratch_types=[
        pltpu.VMEM((CHUNK,), jnp.float32),
        pltpu.VMEM((CHUNK,), jnp.float32),
        pltpu.VMEM((2 * L,), jnp.float32),
        pltpu.VMEM((L,), jnp.float32),
        pltpu.SemaphoreType.DMA,
        pltpu.SemaphoreType.DMA,
    ],
)
def _topk_mean_sc(x_hbm, out_hbm, buf0, buf1, topv, means_v, sem0, sem1):
    cid = lax.axis_index("c")
    sid = lax.axis_index("s")
    wid = sid * NC + cid
    X = _make_idx()

    means_v[...] = jnp.zeros((L,), jnp.float32)
    bufs = (buf0, buf1)
    sems = (sem0, sem1)

    def _run_chunk(buf, thr0):
        def blk_fn(b, thr):
            base = b * (VPB * L)
            vs = [buf[pl.ds(base + j * L, L)] for j in range(VPB)]
            t = vs
            while len(t) > 1:
                t = [jnp.maximum(t[i], t[i + 1]) for i in range(0, len(t), 2)]
            hit = _lane_max(X, t[0])[0] > thr

            def _process(thr_in):
                thr_splat = jnp.full((L,), thr_in, jnp.float32)
                bmv = jnp.zeros((L,), jnp.int32)
                for j in range(VPB):
                    v = buf[pl.ds(base + j * L, L)]
                    bit = (1 << j) if j < 31 else -(1 << 31)
                    bmv = bmv | jnp.where(
                        v > thr_splat, jnp.int32(bit), jnp.int32(0)
                    )
                bm_splat = _lane_or(X, bmv)
                bm = bm_splat[0]
                lane = _lane()
                bits = ((bm_splat >> lane) & 1) + ((bm_splat >> (lane + 16)) & 1)
                cnt = _lane_sum(X, bits)[0]

                int_min = jnp.int32(-(1 << 31))

                def w_body(_, st):
                    bm_c, _thr = st
                    lowbit = bm_c & (-bm_c)
                    fbits = lax.bitcast_convert_type(
                        lowbit.astype(jnp.float32), jnp.int32
                    )
                    j = jnp.where(
                        lowbit == int_min, jnp.int32(31), (fbits >> 23) - 127
                    )
                    v = buf[pl.ds(base + j * L, L)]
                    a0 = topv[pl.ds(0, L)]
                    a1 = topv[pl.ds(L, L)]
                    n0, n1 = _merge_topk(X, a0, a1, v)
                    topv[pl.ds(0, L)] = n0
                    topv[pl.ds(L, L)] = n1
                    return bm_c & (bm_c - 1), n0[0]

                out = lax.fori_loop(0, cnt, w_body, (bm, thr_in))
                return out[1]

            return lax.cond(hit, _process, lambda t: t, thr)

        return lax.fori_loop(0, NBLK, blk_fn, thr0)

    def row_fn(r, carry):
        rowbase = (wid * ROWS_PW + r) * N
        neg = jnp.full((L,), -jnp.inf, jnp.float32)
        topv[pl.ds(0, L)] = neg
        topv[pl.ds(L, L)] = neg

        copies = [None] * NCHUNK
        copies[0] = pltpu.async_copy(
            x_hbm.at[pl.ds(rowbase, CHUNK)], bufs[0], sems[0]
        )
        thr = jnp.float32(-jnp.inf)
        for c in range(NCHUNK):
            copies[c].wait()
            if c + 1 < NCHUNK:
                copies[c + 1] = pltpu.async_copy(
                    x_hbm.at[pl.ds(rowbase + (c + 1) * CHUNK, CHUNK)],
                    bufs[(c + 1) % 2],
                    sems[(c + 1) % 2],
                )
            thr = _run_chunk(bufs[c % 2], thr)

        a0 = topv[pl.ds(0, L)]
        a1 = topv[pl.ds(L, L)]
        mean = _lane_sum(X, a0 + a1) * jnp.float32(1.0 / K_SEL)
        means_v[...] = jnp.where(_lane() == r, mean, means_v[...])
        return carry

    lax.fori_loop(0, ROWS_PW, row_fn, 0)
    pltpu.sync_copy(means_v, out_hbm.at[pl.ds(wid * L, L)])


def kernel(x):
    out = _topk_mean_sc(x.reshape(R * N))
    return out.reshape(NW, L)[:, :ROWS_PW].reshape(R)


# speedup vs baseline: 1.3835x; 1.0047x over previous
import functools

import jax
import jax.numpy as jnp
from jax import lax
from jax.experimental import pallas as pl
from jax.experimental.pallas import tpu as pltpu
from jax.experimental.pallas import tpu_sc as plsc

R = 128
N = 32768
K_SEL = 32
L = 16
NC = 2
NS = 16
NW = NC * NS
ROWS_PW = R // NW
CHUNK = 16384
NCHUNK = N // CHUNK
VPB = 32
NBLK = CHUNK // (L * VPB)

_GDN = lax.GatherDimensionNumbers(
    offset_dims=(), collapsed_slice_dims=(0,), start_index_map=(0,)
)


def _lane():
    return lax.iota(jnp.int32, L)


def _make_idx():
    lane = _lane()
    X = {j: (lane ^ j)[:, None] for j in (1, 2, 4, 8)}
    sortpre = []
    sp = 1
    for p in (2, 4, 8, 16):
        j = p // 2
        sj = sp - 1
        while j >= 1:
            sortpre.append((j, ((lane >> sj) ^ (lane >> sp)) & 1))
            j //= 2
            sj -= 1
        sp += 1
    X["sortpre"] = tuple(sortpre)
    X["mergepre"] = tuple((j, lane & j) for j in (8, 4, 2, 1))
    return X


def _shuffle(X, v, j):
    return lax.gather(
        v, X[j], _GDN, (1,), mode=lax.GatherScatterMode.PROMISE_IN_BOUNDS
    )


def _ce(X, v, j, take_min):
    pv = _shuffle(X, v, j)
    return jnp.where(take_min, jnp.minimum(v, pv), jnp.maximum(v, pv))


def _sort16(X, v):
    for j, pre in X["sortpre"]:
        v = _ce(X, v, j, pre == 0)
    return v


def _bitonic_merge16(X, v):
    for j, pre in X["mergepre"]:
        v = _ce(X, v, j, pre == 0)
    return v


def _lane_max(X, v):
    for j in (8, 4, 2, 1):
        v = jnp.maximum(v, _shuffle(X, v, j))
    return v


def _lane_or(X, v):
    for j in (8, 4, 2, 1):
        v = v | _shuffle(X, v, j)
    return v


def _lane_sum(X, v):
    for j in (8, 4, 2, 1):
        v = v + _shuffle(X, v, j)
    return v


def _merge_topk(X, a0, a1, v):
    rv = lax.rev(_sort16(X, v), (0,))
    mlo = jnp.maximum(a0, rv)
    n0 = jnp.minimum(mlo, a1)
    n1 = jnp.maximum(mlo, a1)
    return _bitonic_merge16(X, n0), _bitonic_merge16(X, n1)


_sc_mesh = plsc.VectorSubcoreMesh(core_axis_name="c", subcore_axis_name="s")


@functools.partial(
    pl.kernel,
    out_type=jax.ShapeDtypeStruct((NW * L,), jnp.float32),
    mesh=_sc_mesh,
    scratch_types=[
        pltpu.VMEM((CHUNK,), jnp.float32),
        pltpu.VMEM((CHUNK,), jnp.float32),
        pltpu.VMEM((2 * L,), jnp.float32),
        pltpu.VMEM((L,), jnp.float32),
        pltpu.SemaphoreType.DMA,
        pltpu.SemaphoreType.DMA,
    ],
)
def _topk_mean_sc(x_hbm, out_hbm, buf0, buf1, topv, means_v, sem0, sem1):
    cid = lax.axis_index("c")
    sid = lax.axis_index("s")
    wid = sid * NC + cid
    X = _make_idx()

    means_v[...] = jnp.zeros((L,), jnp.float32)
    bufs = (buf0, buf1)
    sems = (sem0, sem1)

    def _run_chunk(buf, thr0):
        def blk_fn(b, thr):
            base = b * (VPB * L)
            vs = [buf[pl.ds(base + j * L, L)] for j in range(VPB)]
            t = vs
            while len(t) > 1:
                t = [jnp.maximum(t[i], t[i + 1]) for i in range(0, len(t), 2)]
            hit = _lane_max(X, t[0])[0] > thr

            def _process(thr_in):
                thr_splat = jnp.full((L,), thr_in, jnp.float32)
                bmv = jnp.zeros((L,), jnp.int32)
                for j in range(VPB):
                    v = buf[pl.ds(base + j * L, L)]
                    bit = (1 << j) if j < 31 else -(1 << 31)
                    bmv = bmv | jnp.where(
                        v > thr_splat, jnp.int32(bit), jnp.int32(0)
                    )
                bm_splat = _lane_or(X, bmv)
                bm = bm_splat[0]
                lane = _lane()
                bits = ((bm_splat >> lane) & 1) + ((bm_splat >> (lane + 16)) & 1)
                cnt = _lane_sum(X, bits)[0]

                int_min = jnp.int32(-(1 << 31))

                def w_body(_, st):
                    bm_c, _thr = st
                    lowbit = bm_c & (-bm_c)
                    fbits = lax.bitcast_convert_type(
                        lowbit.astype(jnp.float32), jnp.int32
                    )
                    j = jnp.where(
                        lowbit == int_min, jnp.int32(31), (fbits >> 23) - 127
                    )
                    v = buf[pl.ds(base + j * L, L)]
                    a0 = topv[pl.ds(0, L)]
                    a1 = topv[pl.ds(L, L)]
                    n0, n1 = _merge_topk(X, a0, a1, v)
                    topv[pl.ds(0, L)] = n0
                    topv[pl.ds(L, L)] = n1
                    return bm_c & (bm_c - 1), n0[0]

                out = lax.fori_loop(0, cnt, w_body, (bm, thr_in))
                return out[1]

            return lax.cond(hit, _process, lambda t: t, thr)

        return lax.fori_loop(0, NBLK, blk_fn, thr0)

    def row_fn(r, carry):
        rowbase = (wid * ROWS_PW + r) * N
        neg = jnp.full((L,), -jnp.inf, jnp.float32)
        topv[pl.ds(0, L)] = neg
        topv[pl.ds(L, L)] = neg

        copies = [None] * NCHUNK
        copies[0] = pltpu.async_copy(
            x_hbm.at[pl.ds(rowbase, CHUNK)], bufs[0], sems[0]
        )
        thr = jnp.float32(-jnp.inf)
        for c in range(NCHUNK):
            copies[c].wait()
            if c + 1 < NCHUNK:
                copies[c + 1] = pltpu.async_copy(
                    x_hbm.at[pl.ds(rowbase + (c + 1) * CHUNK, CHUNK)],
                    bufs[(c + 1) % 2],
                    sems[(c + 1) % 2],
                )
            thr = _run_chunk(bufs[c % 2], thr)

        a0 = topv[pl.ds(0, L)]
        a1 = topv[pl.ds(L, L)]
        mean = _lane_sum(X, a0 + a1) * jnp.float32(1.0 / K_SEL)
        means_v[...] = jnp.where(_lane() == r, mean, means_v[...])
        return carry

    lax.fori_loop(0, ROWS_PW, row_fn, 0)
    pltpu.sync_copy(means_v, out_hbm.at[pl.ds(wid * L, L)])


def kernel(x):
    out = _topk_mean_sc(x.reshape(R * N))
    return out.reshape(NW, L)[:, :ROWS_PW].reshape(R)
